# Initial kernel scaffold; baseline (speedup 1.0000x reference)
import jax
import jax.numpy as jnp
from jax import lax
from jax.experimental import pallas as pl
from jax.experimental.pallas import tpu as pltpu
from jax.experimental.pallas import tpu_sc as plsc

_D = 768
_L = 50
_B = 4096
_NC = 2
_NS = 16
_NW = _NC * _NS
_BPW = _B // _NW
_CHUNKS = _D // 16


def _pooled_row(rows_v, acc_v):
    init = tuple(rows_v[0, pl.ds(c * 16, 16)] for c in range(_CHUNKS))

    def add_row(l, accs):
        return tuple(accs[c] + rows_v[l, pl.ds(c * 16, 16)]
                     for c in range(_CHUNKS))

    accs = lax.fori_loop(1, _L, add_row, init)
    scale = jnp.float32(1.0 / _L)
    for c in range(_CHUNKS):
        acc_v[pl.ds(c * 16, 16)] = accs[c] * scale


def _body(ids_hbm, table_hbm, out_hbm,
          ids_v, rows_a, rows_b, acc_a, acc_b,
          gsem_a, gsem_b, osem_a, osem_b):
    wid = lax.axis_index("s") * _NC + lax.axis_index("c")
    base = wid * _BPW

    pltpu.sync_copy(ids_hbm.at[pl.ds(base, _BPW)], ids_v)

    pltpu.async_copy(table_hbm.at[ids_v.at[0]], rows_a, gsem_a)
    pltpu.async_copy(table_hbm.at[ids_v.at[1]], rows_b, gsem_b)

    def pair(p, _):
        b0 = 2 * p
        for off, rows_v, acc_v, gsem, osem in (
                (0, rows_a, acc_a, gsem_a, osem_a),
                (1, rows_b, acc_b, gsem_b, osem_b)):
            b = b0 + off
            pltpu.make_async_copy(
                table_hbm.at[ids_v.at[b]], rows_v, gsem).wait()
            @pl.when(b >= 2)
            def _():
                pltpu.make_async_copy(
                    acc_v, out_hbm.at[base + b - 2], osem).wait()
            _pooled_row(rows_v, acc_v)
            pltpu.async_copy(acc_v, out_hbm.at[base + b], osem)
            @pl.when(b + 2 < _BPW)
            def _():
                pltpu.async_copy(
                    table_hbm.at[ids_v.at[b + 2]], rows_v, gsem)
        return 0

    lax.fori_loop(0, _BPW // 2, pair, 0)

    pltpu.make_async_copy(acc_a, out_hbm.at[base + _BPW - 2], osem_a).wait()
    pltpu.make_async_copy(acc_b, out_hbm.at[base + _BPW - 1], osem_b).wait()


_mesh = plsc.VectorSubcoreMesh(core_axis_name="c", subcore_axis_name="s")

_sc_call = pl.kernel(
    _body,
    out_type=jax.ShapeDtypeStruct((_B, _D), jnp.float32),
    mesh=_mesh,
    scratch_types=[
        pltpu.VMEM((_BPW, _L), jnp.int32),
        pltpu.VMEM((_L, _D), jnp.float32),
        pltpu.VMEM((_L, _D), jnp.float32),
        pltpu.VMEM((_D,), jnp.float32),
        pltpu.VMEM((_D,), jnp.float32),
        pltpu.SemaphoreType.DMA,
        pltpu.SemaphoreType.DMA,
        pltpu.SemaphoreType.DMA,
        pltpu.SemaphoreType.DMA,
    ],
)


@jax.jit
def _run(input_ids, table):
    return _sc_call(input_ids, table)


def kernel(input_ids, table):
    return _run(input_ids, table)


# speedup vs baseline: 1.1102x; 1.1102x over previous
import jax
import jax.numpy as jnp
from jax import lax
from jax.experimental import pallas as pl
from jax.experimental.pallas import tpu as pltpu
from jax.experimental.pallas import tpu_sc as plsc

_D = 768
_L = 50
_LP = 56
_B = 4096
_NC = 2
_NS = 16
_NW = _NC * _NS
_BPW = _B // _NW
_CHUNKS = _D // 16


def _pooled_row(rows_v, acc_v):
    init = tuple(rows_v[0, pl.ds(c * 16, 16)] for c in range(_CHUNKS))

    def add_row(l, accs):
        return tuple(accs[c] + rows_v[l, pl.ds(c * 16, 16)]
                     for c in range(_CHUNKS))

    accs = lax.fori_loop(1, _L, add_row, init)
    scale = jnp.float32(1.0 / _L)
    for c in range(_CHUNKS):
        acc_v[pl.ds(c * 16, 16)] = accs[c] * scale


def _body(ids_hbm, table_hbm, out_hbm,
          ids_v, rows_a, rows_b, acc_a, acc_b,
          gsem_a, gsem_b, osem_a, osem_b):
    wid = lax.axis_index("s") * _NC + lax.axis_index("c")
    base = wid * _BPW

    pltpu.sync_copy(ids_hbm.at[pl.ds(base * _LP, _BPW * _LP)], ids_v)

    _PIECES = ((0, 48), (48, 8))

    def gather_row(b, rows_v, gsem):
        for s, n in _PIECES:
            pltpu.async_copy(
                table_hbm.at[ids_v.at[pl.ds(b * _LP + s, n)]],
                rows_v.at[pl.ds(s, n)], gsem)

    def wait_row(b, rows_v, gsem):
        for s, n in _PIECES:
            pltpu.make_async_copy(
                table_hbm.at[ids_v.at[pl.ds(b * _LP + s, n)]],
                rows_v.at[pl.ds(s, n)], gsem).wait()

    gather_row(0, rows_a, gsem_a)
    gather_row(1, rows_b, gsem_b)

    def pair(p, _):
        b0 = 2 * p
        for off, rows_v, acc_v, gsem, osem in (
                (0, rows_a, acc_a, gsem_a, osem_a),
                (1, rows_b, acc_b, gsem_b, osem_b)):
            b = b0 + off
            wait_row(b, rows_v, gsem)
            @pl.when(b >= 2)
            def _():
                pltpu.make_async_copy(
                    acc_v, out_hbm.at[base + b - 2], osem).wait()
            _pooled_row(rows_v, acc_v)
            pltpu.async_copy(acc_v, out_hbm.at[base + b], osem)
            @pl.when(b + 2 < _BPW)
            def _():
                gather_row(b + 2, rows_v, gsem)
        return 0

    lax.fori_loop(0, _BPW // 2, pair, 0)

    pltpu.make_async_copy(acc_a, out_hbm.at[base + _BPW - 2], osem_a).wait()
    pltpu.make_async_copy(acc_b, out_hbm.at[base + _BPW - 1], osem_b).wait()


_mesh = plsc.VectorSubcoreMesh(core_axis_name="c", subcore_axis_name="s")

_sc_call = pl.kernel(
    _body,
    out_type=jax.ShapeDtypeStruct((_B, _D), jnp.float32),
    mesh=_mesh,
    scratch_types=[
        pltpu.VMEM((_BPW * _LP,), jnp.int32),
        pltpu.VMEM((_LP, _D), jnp.float32),
        pltpu.VMEM((_LP, _D), jnp.float32),
        pltpu.VMEM((_D,), jnp.float32),
        pltpu.VMEM((_D,), jnp.float32),
        pltpu.SemaphoreType.DMA,
        pltpu.SemaphoreType.DMA,
        pltpu.SemaphoreType.DMA,
        pltpu.SemaphoreType.DMA,
    ],
)


@jax.jit
def _run(input_ids, table):
    ids_flat = jnp.pad(input_ids, ((0, 0), (0, _LP - _L))).reshape(-1)
    return _sc_call(ids_flat, table)


def kernel(input_ids, table):
    return _run(input_ids, table)
